# baseline (speedup 1.0000x reference)
import functools

import jax
import jax.numpy as jnp
from jax import lax
from jax.experimental import pallas as pl
from jax.experimental.pallas import tpu as pltpu
from jax.experimental.pallas import tpu_sc as plsc

N = 10000
E = 320000
D = 128
H1 = 32
H2 = 16

NC = 2
NS = 16
NW = NC * NS
CH = 128
EPW = E // NW
GK = 8
NG = 10
CPW = GK * NG
EPW_PAD = CPW * CH
NP = 10112
RPT = NP // NS

_f32 = jnp.float32


def _make_sc_agg(H):
    mesh = plsc.VectorSubcoreMesh(
        core_axis_name="c", subcore_axis_name="s",
        num_cores=NC, num_subcores=NS)

    def body(src3, dst3, table, zeros, out, src_v, dst_v, rows_v, acc_sh,
             gsem, ssem):
        cid = lax.axis_index("c")
        sid = lax.axis_index("s")
        wid = sid * NC + cid
        r0 = sid * RPT
        pltpu.sync_copy(zeros.at[pl.ds(r0, RPT)], acc_sh.at[pl.ds(r0, RPT)])
        pltpu.sync_copy(src3.at[wid], src_v)
        pltpu.sync_copy(dst3.at[wid], dst_v)
        plsc.subcore_barrier()

        def group(g, carry):
            j0 = g * GK
            for b in range(GK):
                pltpu.async_copy(table.at[src_v.at[j0 + b]], rows_v.at[b],
                                 gsem)
            for b in range(GK):
                pltpu.make_async_copy(table.at[src_v.at[j0 + b]],
                                      rows_v.at[b], gsem).wait()
            for b in range(GK):
                pltpu.async_copy(rows_v.at[b], acc_sh.at[dst_v.at[j0 + b]],
                                 ssem, add=True)
            for b in range(GK):
                pltpu.make_async_copy(rows_v.at[b],
                                      acc_sh.at[dst_v.at[j0 + b]],
                                      ssem).wait()
            return carry

        lax.fori_loop(0, NG, group, 0)
        plsc.subcore_barrier()
        pltpu.sync_copy(acc_sh.at[pl.ds(r0, RPT)],
                        out.at[cid, pl.ds(r0, RPT)])

    return pl.kernel(
        body,
        out_type=jax.ShapeDtypeStruct((NC, NP, H), _f32),
        mesh=mesh,
        compiler_params=pltpu.CompilerParams(use_tc_tiling_on_sc=False),
        scratch_types=[
            pltpu.VMEM((CPW, CH), jnp.int32),
            pltpu.VMEM((CPW, CH), jnp.int32),
            pltpu.VMEM((GK, CH, H), _f32),
            pltpu.VMEM_SHARED((NP, H), _f32),
            pltpu.SemaphoreType.DMA,
            pltpu.SemaphoreType.DMA,
        ],
    )


_sc_agg8 = _make_sc_agg(8)
_sc_agg32 = _make_sc_agg(H1)
_sc_agg16 = _make_sc_agg(H2)


def _mm1_body(x_ref, w1_ref, hw_ref):
    hw_ref[...] = jnp.dot(x_ref[...], w1_ref[...],
                          preferred_element_type=_f32)


def _scale1_body(degp_ref, hwr_ref, hw_ref, norm_ref):
    degsum = degp_ref[0, :, 0:1] + degp_ref[1, :, 0:1]
    norm = lax.rsqrt(degsum + 1.0)
    hw_ref[:N, :] = hwr_ref[...] * norm[:N]
    hw_ref[N:, :] = jnp.zeros((NP - N, H1), _f32)
    norm_ref[...] = norm[:N]


def _prep2_body(p_ref, hwp_ref, norm_ref, w2_ref, out_ref):
    agg = p_ref[0, :N, :] + p_ref[1, :N, :] + hwp_ref[:N, :]
    h1 = jnp.maximum(agg * norm_ref[...], 0.0)
    hw2 = jnp.dot(h1, w2_ref[...], preferred_element_type=_f32)
    out_ref[:N, :] = hw2 * norm_ref[...]
    out_ref[N:, :] = jnp.zeros((NP - N, H2), _f32)


BR = 400


def _decode_body(pb_ref, hwpb_ref, normb_ref, pf_ref, hwpf_ref,
                 normf_ref, out_ref):
    zb = (pb_ref[0] + pb_ref[1] + hwpb_ref[...]) * normb_ref[...]
    aggf = pf_ref[0, :N, :] + pf_ref[1, :N, :] + hwpf_ref[:N, :]
    zf = aggf * normf_ref[...]
    out_ref[...] = lax.dot_general(
        zb, zf, (((1,), (1,)), ((), ())), preferred_element_type=_f32)


def kernel(x, edge_index, W1, W2):
    src = edge_index[0]
    dst = edge_index[1]
    pad = ((0, 0), (0, EPW_PAD - EPW))
    src3 = jnp.pad(src.reshape(NW, EPW), pad, constant_values=N)
    src3 = src3.reshape(NW, CPW, CH)
    dst3 = jnp.pad(dst.reshape(NW, EPW), pad, constant_values=N)
    dst3 = dst3.reshape(NW, CPW, CH)

    ones8 = jnp.zeros((NP, 8), _f32).at[:N].set(1.0)
    z8 = jnp.zeros((NP, 8), _f32)
    z32 = jnp.zeros((NP, H1), _f32)
    z16 = jnp.zeros((NP, H2), _f32)

    degp = _sc_agg8(src3, dst3, ones8, z8)

    hw1r = pl.pallas_call(
        _mm1_body,
        out_shape=jax.ShapeDtypeStruct((N, H1), _f32),
    )(x, W1)

    hw1p, norm = pl.pallas_call(
        _scale1_body,
        out_shape=(jax.ShapeDtypeStruct((NP, H1), _f32),
                   jax.ShapeDtypeStruct((N, 1), _f32)),
    )(degp, hw1r)

    p1 = _sc_agg32(src3, dst3, hw1p, z32)

    hw2p = pl.pallas_call(
        _prep2_body,
        out_shape=jax.ShapeDtypeStruct((NP, H2), _f32),
    )(p1, hw1p, norm, W2)

    p2 = _sc_agg16(src3, dst3, hw2p, z16)

    recon = pl.pallas_call(
        _decode_body,
        grid=(N // BR,),
        in_specs=[
            pl.BlockSpec((NC, BR, H2), lambda i: (0, i, 0)),
            pl.BlockSpec((BR, H2), lambda i: (i, 0)),
            pl.BlockSpec((BR, 1), lambda i: (i, 0)),
            pl.BlockSpec((NC, NP, H2), lambda i: (0, 0, 0)),
            pl.BlockSpec((NP, H2), lambda i: (0, 0)),
            pl.BlockSpec((N, 1), lambda i: (0, 0)),
        ],
        out_specs=pl.BlockSpec((BR, N), lambda i: (i, 0)),
        out_shape=jax.ShapeDtypeStruct((N, N), _f32),
    )(p2, hw2p, norm, p2, hw2p, norm)

    return recon.reshape(-1)


# speedup vs baseline: 1.0693x; 1.0693x over previous
import functools

import jax
import jax.numpy as jnp
from jax import lax
from jax.experimental import pallas as pl
from jax.experimental.pallas import tpu as pltpu
from jax.experimental.pallas import tpu_sc as plsc

N = 10000
E = 320000
D = 128
H1 = 32
H2 = 16

NC = 2
NS = 16
NW = NC * NS
CH = 128
EPW = E // NW
GK = 8
NG = 10
CPW = GK * NG
EPW_PAD = CPW * CH
NP = 10112
RPT = NP // NS

_f32 = jnp.float32


def _make_sc_agg(H, gather=True):
    mesh = plsc.VectorSubcoreMesh(
        core_axis_name="c", subcore_axis_name="s",
        num_cores=NC, num_subcores=NS)

    def body(*args):
        if gather:
            (src3, dst3, table, zeros, out,
             src_v, dst_v, rows_v, acc_sh, gsem, ssem) = args
        else:
            (dst3, table, zeros, out,
             dst_v, rows_v, acc_sh, gsem, ssem) = args
        cid = lax.axis_index("c")
        sid = lax.axis_index("s")
        wid = sid * NC + cid
        r0 = sid * RPT
        pltpu.sync_copy(zeros.at[pl.ds(r0, RPT), pl.ds(0, H)],
                        acc_sh.at[pl.ds(r0, RPT)])
        if gather:
            pltpu.sync_copy(src3.at[wid], src_v)
        else:
            pltpu.sync_copy(table, rows_v.at[0])
        pltpu.sync_copy(dst3.at[wid], dst_v)
        plsc.subcore_barrier()

        def group(g, carry):
            j0 = g * GK
            if gather:
                for b in range(GK):
                    pltpu.async_copy(table.at[src_v.at[j0 + b]], rows_v.at[b],
                                     gsem)
                for b in range(GK):
                    pltpu.make_async_copy(table.at[src_v.at[j0 + b]],
                                          rows_v.at[b], gsem).wait()
            for b in range(GK):
                pltpu.async_copy(rows_v.at[b if gather else 0],
                                 acc_sh.at[dst_v.at[j0 + b]],
                                 ssem, add=True)
            for b in range(GK):
                pltpu.make_async_copy(rows_v.at[b if gather else 0],
                                      acc_sh.at[dst_v.at[j0 + b]],
                                      ssem).wait()
            return carry

        lax.fori_loop(0, NG, group, 0)
        plsc.subcore_barrier()
        pltpu.sync_copy(acc_sh.at[pl.ds(r0, RPT)],
                        out.at[cid, pl.ds(r0, RPT)])

    return pl.kernel(
        body,
        out_type=jax.ShapeDtypeStruct((NC, NP, H), _f32),
        mesh=mesh,
        compiler_params=pltpu.CompilerParams(use_tc_tiling_on_sc=False),
        scratch_types=(
            [pltpu.VMEM((CPW, CH), jnp.int32)] if gather else []) + [
            pltpu.VMEM((CPW, CH), jnp.int32),
            pltpu.VMEM((GK, CH, H), _f32),
            pltpu.VMEM_SHARED((NP, H), _f32),
            pltpu.SemaphoreType.DMA,
            pltpu.SemaphoreType.DMA,
        ],
    )


_sc_deg = _make_sc_agg(8, gather=False)
_sc_agg32 = _make_sc_agg(H1)
_sc_agg16 = _make_sc_agg(H2)


def _mm1_body(x_ref, w1_ref, hw_ref):
    hw_ref[...] = jnp.dot(x_ref[...], w1_ref[...],
                          preferred_element_type=_f32)


def _scale1_body(degp_ref, hwr_ref, hw_ref, norm_ref):
    degsum = degp_ref[0, :, 0:1] + degp_ref[1, :, 0:1]
    norm = lax.rsqrt(degsum + 1.0)
    hw_ref[:N, :] = hwr_ref[...] * norm[:N]
    hw_ref[N:, :] = jnp.zeros((NP - N, H1), _f32)
    norm_ref[...] = norm[:N]


def _prep2_body(p_ref, hwp_ref, norm_ref, w2_ref, out_ref):
    agg = p_ref[0, :N, :] + p_ref[1, :N, :] + hwp_ref[:N, :]
    h1 = jnp.maximum(agg * norm_ref[...], 0.0)
    hw2 = jnp.dot(h1, w2_ref[...], preferred_element_type=_f32)
    out_ref[:N, :] = hw2 * norm_ref[...]
    out_ref[N:, :] = jnp.zeros((NP - N, H2), _f32)


BR = 400


def _decode_body(pb_ref, hwpb_ref, normb_ref, pf_ref, hwpf_ref,
                 normf_ref, out_ref):
    zb = (pb_ref[0] + pb_ref[1] + hwpb_ref[...]) * normb_ref[...]
    aggf = pf_ref[0, :N, :] + pf_ref[1, :N, :] + hwpf_ref[:N, :]
    zf = aggf * normf_ref[...]
    out_ref[...] = lax.dot_general(
        zb, zf, (((1,), (1,)), ((), ())), preferred_element_type=_f32)


def kernel(x, edge_index, W1, W2):
    src = edge_index[0]
    dst = edge_index[1]
    pad = ((0, 0), (0, EPW_PAD - EPW))
    src3 = jnp.pad(src.reshape(NW, EPW), pad, constant_values=N)
    src3 = src3.reshape(NW, CPW, CH)
    dst3 = jnp.pad(dst.reshape(NW, EPW), pad, constant_values=N)
    dst3 = dst3.reshape(NW, CPW, CH)

    ones_ch = jnp.ones((CH, 8), _f32)
    zeros32 = jnp.zeros((NP, H1), _f32)

    degp = _sc_deg(dst3, ones_ch, zeros32)

    hw1r = pl.pallas_call(
        _mm1_body,
        out_shape=jax.ShapeDtypeStruct((N, H1), _f32),
    )(x, W1)

    hw1p, norm = pl.pallas_call(
        _scale1_body,
        out_shape=(jax.ShapeDtypeStruct((NP, H1), _f32),
                   jax.ShapeDtypeStruct((N, 1), _f32)),
    )(degp, hw1r)

    p1 = _sc_agg32(src3, dst3, hw1p, zeros32)

    hw2p = pl.pallas_call(
        _prep2_body,
        out_shape=jax.ShapeDtypeStruct((NP, H2), _f32),
    )(p1, hw1p, norm, W2)

    p2 = _sc_agg16(src3, dst3, hw2p, zeros32)

    recon = pl.pallas_call(
        _decode_body,
        grid=(N // BR,),
        in_specs=[
            pl.BlockSpec((NC, BR, H2), lambda i: (0, i, 0)),
            pl.BlockSpec((BR, H2), lambda i: (i, 0)),
            pl.BlockSpec((BR, 1), lambda i: (i, 0)),
            pl.BlockSpec((NC, NP, H2), lambda i: (0, 0, 0)),
            pl.BlockSpec((NP, H2), lambda i: (0, 0)),
            pl.BlockSpec((N, 1), lambda i: (0, 0)),
        ],
        out_specs=pl.BlockSpec((BR, N), lambda i: (i, 0)),
        out_shape=jax.ShapeDtypeStruct((N, N), _f32),
    )(p2, hw2p, norm, p2, hw2p, norm)

    return recon.reshape(-1)


# speedup vs baseline: 1.0771x; 1.0072x over previous
import functools

import jax
import jax.numpy as jnp
from jax import lax
from jax.experimental import pallas as pl
from jax.experimental.pallas import tpu as pltpu
from jax.experimental.pallas import tpu_sc as plsc

N = 10000
E = 320000
D = 128
H1 = 32
H2 = 16

NC = 2
NS = 16
NW = NC * NS
CH = 128
EPW = E // NW
GK = 16
NG = 5
CPW = GK * NG
EPW_PAD = CPW * CH
NP = 10112
RPT = NP // NS

_f32 = jnp.float32


def _make_sc_agg(H, gather=True):
    mesh = plsc.VectorSubcoreMesh(
        core_axis_name="c", subcore_axis_name="s",
        num_cores=NC, num_subcores=NS)

    def body(*args):
        if gather:
            (src3, dst3, table, zeros, out,
             src_v, dst_v, rows_v, acc_sh, gsem, ssem) = args
        else:
            (dst3, table, zeros, out,
             dst_v, rows_v, acc_sh, gsem, ssem) = args
        cid = lax.axis_index("c")
        sid = lax.axis_index("s")
        wid = sid * NC + cid
        r0 = sid * RPT
        pltpu.sync_copy(zeros.at[pl.ds(r0, RPT), pl.ds(0, H)],
                        acc_sh.at[pl.ds(r0, RPT)])
        if gather:
            pltpu.sync_copy(src3.at[wid], src_v)
        else:
            pltpu.sync_copy(table, rows_v.at[0])
        pltpu.sync_copy(dst3.at[wid], dst_v)
        plsc.subcore_barrier()

        def group(g, carry):
            j0 = g * GK
            if gather:
                for b in range(GK):
                    pltpu.async_copy(table.at[src_v.at[j0 + b]], rows_v.at[b],
                                     gsem)
                for b in range(GK):
                    pltpu.make_async_copy(table.at[src_v.at[j0 + b]],
                                          rows_v.at[b], gsem).wait()
            for b in range(GK):
                pltpu.async_copy(rows_v.at[b if gather else 0],
                                 acc_sh.at[dst_v.at[j0 + b]],
                                 ssem, add=True)
            for b in range(GK):
                pltpu.make_async_copy(rows_v.at[b if gather else 0],
                                      acc_sh.at[dst_v.at[j0 + b]],
                                      ssem).wait()
            return carry

        lax.fori_loop(0, NG, group, 0)
        plsc.subcore_barrier()
        pltpu.sync_copy(acc_sh.at[pl.ds(r0, RPT)],
                        out.at[cid, pl.ds(r0, RPT)])

    return pl.kernel(
        body,
        out_type=jax.ShapeDtypeStruct((NC, NP, H), _f32),
        mesh=mesh,
        compiler_params=pltpu.CompilerParams(use_tc_tiling_on_sc=False),
        scratch_types=(
            [pltpu.VMEM((CPW, CH), jnp.int32)] if gather else []) + [
            pltpu.VMEM((CPW, CH), jnp.int32),
            pltpu.VMEM((GK, CH, H), _f32),
            pltpu.VMEM_SHARED((NP, H), _f32),
            pltpu.SemaphoreType.DMA,
            pltpu.SemaphoreType.DMA,
        ],
    )


_sc_deg = _make_sc_agg(8, gather=False)
_sc_agg32 = _make_sc_agg(H1)
_sc_agg16 = _make_sc_agg(H2)


def _mm1_body(x_ref, w1_ref, hw_ref):
    hw_ref[...] = jnp.dot(x_ref[...], w1_ref[...],
                          preferred_element_type=_f32)


def _scale1_body(degp_ref, hwr_ref, hw_ref, norm_ref):
    degsum = degp_ref[0, :, 0:1] + degp_ref[1, :, 0:1]
    norm = lax.rsqrt(degsum + 1.0)
    hw_ref[:N, :] = hwr_ref[...] * norm[:N]
    hw_ref[N:, :] = jnp.zeros((NP - N, H1), _f32)
    norm_ref[...] = norm[:N]


def _prep2_body(p_ref, hwp_ref, norm_ref, w2_ref, out_ref):
    agg = p_ref[0, :N, :] + p_ref[1, :N, :] + hwp_ref[:N, :]
    h1 = jnp.maximum(agg * norm_ref[...], 0.0)
    hw2 = jnp.dot(h1, w2_ref[...], preferred_element_type=_f32)
    out_ref[:N, :] = hw2 * norm_ref[...]
    out_ref[N:, :] = jnp.zeros((NP - N, H2), _f32)


BR = 400


def _decode_body(pb_ref, hwpb_ref, normb_ref, pf_ref, hwpf_ref,
                 normf_ref, out_ref):
    zb = (pb_ref[0] + pb_ref[1] + hwpb_ref[...]) * normb_ref[...]
    aggf = pf_ref[0, :N, :] + pf_ref[1, :N, :] + hwpf_ref[:N, :]
    zf = aggf * normf_ref[...]
    out_ref[...] = lax.dot_general(
        zb, zf, (((1,), (1,)), ((), ())), preferred_element_type=_f32)


def kernel(x, edge_index, W1, W2):
    src = edge_index[0]
    dst = edge_index[1]
    pad = ((0, 0), (0, EPW_PAD - EPW))
    src3 = jnp.pad(src.reshape(NW, EPW), pad, constant_values=N)
    src3 = src3.reshape(NW, CPW, CH)
    dst3 = jnp.pad(dst.reshape(NW, EPW), pad, constant_values=N)
    dst3 = dst3.reshape(NW, CPW, CH)

    ones_ch = jnp.ones((CH, 8), _f32)
    zeros32 = jnp.zeros((NP, H1), _f32)

    degp = _sc_deg(dst3, ones_ch, zeros32)

    hw1r = pl.pallas_call(
        _mm1_body,
        out_shape=jax.ShapeDtypeStruct((N, H1), _f32),
    )(x, W1)

    hw1p, norm = pl.pallas_call(
        _scale1_body,
        out_shape=(jax.ShapeDtypeStruct((NP, H1), _f32),
                   jax.ShapeDtypeStruct((N, 1), _f32)),
    )(degp, hw1r)

    p1 = _sc_agg32(src3, dst3, hw1p, zeros32)

    hw2p = pl.pallas_call(
        _prep2_body,
        out_shape=jax.ShapeDtypeStruct((NP, H2), _f32),
    )(p1, hw1p, norm, W2)

    p2 = _sc_agg16(src3, dst3, hw2p, zeros32)

    recon = pl.pallas_call(
        _decode_body,
        grid=(N // BR,),
        in_specs=[
            pl.BlockSpec((NC, BR, H2), lambda i: (0, i, 0)),
            pl.BlockSpec((BR, H2), lambda i: (i, 0)),
            pl.BlockSpec((BR, 1), lambda i: (i, 0)),
            pl.BlockSpec((NC, NP, H2), lambda i: (0, 0, 0)),
            pl.BlockSpec((NP, H2), lambda i: (0, 0)),
            pl.BlockSpec((N, 1), lambda i: (0, 0)),
        ],
        out_specs=pl.BlockSpec((BR, N), lambda i: (i, 0)),
        out_shape=jax.ShapeDtypeStruct((N, N), _f32),
    )(p2, hw2p, norm, p2, hw2p, norm)

    return recon.reshape(-1)


# speedup vs baseline: 1.2491x; 1.1598x over previous
import functools

import jax
import jax.numpy as jnp
from jax import lax
from jax.experimental import pallas as pl
from jax.experimental.pallas import tpu as pltpu
from jax.experimental.pallas import tpu_sc as plsc

N = 10000
E = 320000
D = 128
H1 = 32
H2 = 16

NC = 2
NS = 16
NW = NC * NS
CH = 128
EPW = E // NW
GK = 16
NG = 5
CPW = GK * NG
EPW_PAD = CPW * CH
NP = 10112
RPT = NP // NS

_f32 = jnp.float32


def _make_sc_agg(H, gather=True):
    mesh = plsc.VectorSubcoreMesh(
        core_axis_name="c", subcore_axis_name="s",
        num_cores=NC, num_subcores=NS)

    def body(*args):
        if gather:
            (src3, dst3, table, zeros, out,
             src_v, dst_v, rows_v, acc_sh, tab_sh, gsem, ssem) = args
        else:
            (dst3, table, zeros, out,
             dst_v, rows_v, acc_sh, gsem, ssem) = args
        cid = lax.axis_index("c")
        sid = lax.axis_index("s")
        wid = sid * NC + cid
        r0 = sid * RPT
        pltpu.sync_copy(zeros.at[pl.ds(r0, RPT), pl.ds(0, H)],
                        acc_sh.at[pl.ds(r0, RPT)])
        if gather:
            pltpu.sync_copy(src3.at[wid], src_v)
            pltpu.sync_copy(table.at[pl.ds(r0, RPT)],
                            tab_sh.at[pl.ds(r0, RPT)])
        else:
            pltpu.sync_copy(table, rows_v.at[0])
        pltpu.sync_copy(dst3.at[wid], dst_v)
        plsc.subcore_barrier()

        def group(g, carry):
            j0 = g * GK
            if gather:
                for b in range(GK):
                    pltpu.async_copy(tab_sh.at[src_v.at[j0 + b]],
                                     rows_v.at[b], gsem)
                for b in range(GK):
                    pltpu.make_async_copy(tab_sh.at[src_v.at[j0 + b]],
                                          rows_v.at[b], gsem).wait()
            for b in range(GK):
                pltpu.async_copy(rows_v.at[b if gather else 0],
                                 acc_sh.at[dst_v.at[j0 + b]],
                                 ssem, add=True)
            for b in range(GK):
                pltpu.make_async_copy(rows_v.at[b if gather else 0],
                                      acc_sh.at[dst_v.at[j0 + b]],
                                      ssem).wait()
            return carry

        lax.fori_loop(0, NG, group, 0)
        plsc.subcore_barrier()
        pltpu.sync_copy(acc_sh.at[pl.ds(r0, RPT)],
                        out.at[cid, pl.ds(r0, RPT)])

    return pl.kernel(
        body,
        out_type=jax.ShapeDtypeStruct((NC, NP, H), _f32),
        mesh=mesh,
        compiler_params=pltpu.CompilerParams(use_tc_tiling_on_sc=False),
        scratch_types=(
            [pltpu.VMEM((CPW, CH), jnp.int32)] if gather else []) + [
            pltpu.VMEM((CPW, CH), jnp.int32),
            pltpu.VMEM((GK, CH, H), _f32),
            pltpu.VMEM_SHARED((NP, H), _f32),
        ] + ([pltpu.VMEM_SHARED((NP, H), _f32)] if gather else []) + [
            pltpu.SemaphoreType.DMA,
            pltpu.SemaphoreType.DMA,
        ],
    )


_sc_deg = _make_sc_agg(8, gather=False)
_sc_agg32 = _make_sc_agg(H1)
_sc_agg16 = _make_sc_agg(H2)


def _mm1_body(x_ref, w1_ref, hw_ref):
    hw_ref[...] = jnp.dot(x_ref[...], w1_ref[...],
                          preferred_element_type=_f32)


def _scale1_body(degp_ref, hwr_ref, hw_ref, norm_ref):
    degsum = degp_ref[0, :, 0:1] + degp_ref[1, :, 0:1]
    norm = lax.rsqrt(degsum + 1.0)
    hw_ref[:N, :] = hwr_ref[...] * norm[:N]
    hw_ref[N:, :] = jnp.zeros((NP - N, H1), _f32)
    norm_ref[...] = norm[:N]


def _prep2_body(p_ref, hwp_ref, norm_ref, w2_ref, out_ref):
    agg = p_ref[0, :N, :] + p_ref[1, :N, :] + hwp_ref[:N, :]
    h1 = jnp.maximum(agg * norm_ref[...], 0.0)
    hw2 = jnp.dot(h1, w2_ref[...], preferred_element_type=_f32)
    out_ref[:N, :] = hw2 * norm_ref[...]
    out_ref[N:, :] = jnp.zeros((NP - N, H2), _f32)


BR = 400


def _decode_body(pb_ref, hwpb_ref, normb_ref, pf_ref, hwpf_ref,
                 normf_ref, out_ref):
    zb = (pb_ref[0] + pb_ref[1] + hwpb_ref[...]) * normb_ref[...]
    aggf = pf_ref[0, :N, :] + pf_ref[1, :N, :] + hwpf_ref[:N, :]
    zf = aggf * normf_ref[...]
    out_ref[...] = lax.dot_general(
        zb, zf, (((1,), (1,)), ((), ())), preferred_element_type=_f32)


def kernel(x, edge_index, W1, W2):
    src = edge_index[0]
    dst = edge_index[1]
    pad = ((0, 0), (0, EPW_PAD - EPW))
    src3 = jnp.pad(src.reshape(NW, EPW), pad, constant_values=N)
    src3 = src3.reshape(NW, CPW, CH)
    dst3 = jnp.pad(dst.reshape(NW, EPW), pad, constant_values=N)
    dst3 = dst3.reshape(NW, CPW, CH)

    ones_ch = jnp.ones((CH, 8), _f32)
    zeros32 = jnp.zeros((NP, H1), _f32)

    degp = _sc_deg(dst3, ones_ch, zeros32)

    hw1r = pl.pallas_call(
        _mm1_body,
        out_shape=jax.ShapeDtypeStruct((N, H1), _f32),
    )(x, W1)

    hw1p, norm = pl.pallas_call(
        _scale1_body,
        out_shape=(jax.ShapeDtypeStruct((NP, H1), _f32),
                   jax.ShapeDtypeStruct((N, 1), _f32)),
    )(degp, hw1r)

    p1 = _sc_agg32(src3, dst3, hw1p, zeros32)

    hw2p = pl.pallas_call(
        _prep2_body,
        out_shape=jax.ShapeDtypeStruct((NP, H2), _f32),
    )(p1, hw1p, norm, W2)

    p2 = _sc_agg16(src3, dst3, hw2p, zeros32)

    recon = pl.pallas_call(
        _decode_body,
        grid=(N // BR,),
        in_specs=[
            pl.BlockSpec((NC, BR, H2), lambda i: (0, i, 0)),
            pl.BlockSpec((BR, H2), lambda i: (i, 0)),
            pl.BlockSpec((BR, 1), lambda i: (i, 0)),
            pl.BlockSpec((NC, NP, H2), lambda i: (0, 0, 0)),
            pl.BlockSpec((NP, H2), lambda i: (0, 0)),
            pl.BlockSpec((N, 1), lambda i: (0, 0)),
        ],
        out_specs=pl.BlockSpec((BR, N), lambda i: (i, 0)),
        out_shape=jax.ShapeDtypeStruct((N, N), _f32),
    )(p2, hw2p, norm, p2, hw2p, norm)

    return recon.reshape(-1)
